# baseline (speedup 1.0000x reference)
import functools

import jax
import jax.numpy as jnp
from jax import lax
from jax.experimental import pallas as pl
from jax.experimental.pallas import tpu as pltpu
from jax.experimental.pallas import tpu_sc as plsc

_B = 4096
_L = 200
_D = 512
_CH = 40
_NCH = _L // _CH
_NC = 2
_NS = 16
_NW = _NC * _NS
_BPW = _B // _NW
_LANES = 16
_NG = 3
_NO = 2


def _body(idx_hbm, pos_hbm, table_hbm, out_hbm, idx_v, pos_v,
          gbuf0, gbuf1, gbuf2, obuf0, obuf1,
          gsem0, gsem1, gsem2, wsem0, wsem1):
    c = lax.axis_index("c")
    s = lax.axis_index("s")
    wid = s * _NC + c
    base = wid * _BPW
    gbufs = (gbuf0, gbuf1, gbuf2)
    obufs = (obuf0, obuf1)
    gsems = (gsem0, gsem1, gsem2)
    wsems = (wsem0, wsem1)

    def start_gather(bl, p):
        pltpu.async_copy(
            table_hbm.at[idx_v.at[pl.ds(bl * _CH, _CH)]], gbufs[p], gsems[p])

    def wait_gather(bl, p):
        pltpu.make_async_copy(
            table_hbm.at[idx_v.at[pl.ds(bl * _CH, _CH)]], gbufs[p],
            gsems[p]).wait()

    def out_slice(bl, ch):
        row0 = (base + bl) * _L + ch * _CH
        return out_hbm.at[pl.ds(row0, _CH)]

    def convert_add(gp, op):
        gbuf, obuf = gbufs[gp], obufs[op]
        shift = jnp.full((_LANES,), 16, dtype=jnp.int32)
        himask = jnp.full((_LANES,), -65536, dtype=jnp.int32)

        def r_body(r, _):
            for jj in range(_D // (2 * _LANES)):
                u = gbuf[r, pl.ds(jj * _LANES, _LANES)]
                lo = lax.bitcast_convert_type(u << shift, jnp.float32)
                hi = lax.bitcast_convert_type(u & himask, jnp.float32)
                sl0 = pl.ds(jj * 2 * _LANES, _LANES)
                sl1 = pl.ds(jj * 2 * _LANES + _LANES, _LANES)
                obuf[r, sl0] = lo + pos_v[r, sl0]
                obuf[r, sl1] = hi + pos_v[r, sl1]
            return 0

        lax.fori_loop(0, _CH, r_body, 0)

    def start_write(bl, op, ch):
        pltpu.async_copy(obufs[op], out_slice(bl, ch), wsems[op])

    def wait_write(bl, op, ch):
        pltpu.make_async_copy(obufs[op], out_slice(bl, ch), wsems[op]).wait()

    def ch_body(ch, _):
        pltpu.sync_copy(
            idx_hbm.at[pl.ds(ch * _B * _CH + base * _CH, _BPW * _CH)], idx_v)
        pltpu.sync_copy(pos_hbm.at[pl.ds(ch * _CH, _CH)], pos_v)

        start_gather(0, 0)
        start_gather(1, 1)

        for bl in (0, 1):
            wait_gather(bl, bl)
            start_gather(bl + 2, bl + 2 - _NG * ((bl + 2) // _NG))
            convert_add(bl, bl)
            start_write(bl, bl, ch)

        def six_body(i, _):
            for k in range(6):
                bl = 6 * i + 2 + k
                gp = (2 + k) % _NG
                gq = (4 + k) % _NG
                op = k % _NO
                wait_gather(bl, gp)

                @pl.when(bl < _BPW - 2)
                def _():
                    start_gather(bl + 2, gq)

                wait_write(bl - 2, op, ch)
                convert_add(gp, op)
                start_write(bl, op, ch)
            return 0

        lax.fori_loop(0, (_BPW - 2) // 6, six_body, 0)

        wait_write(_BPW - 2, 0, ch)
        wait_write(_BPW - 1, 1, ch)
        return 0

    lax.fori_loop(0, _NCH, ch_body, 0)


@jax.jit
def kernel(inputs, table, pos_embed):
    idx_r = (inputs.astype(jnp.int32)
             .reshape(_B, _NCH, _CH)
             .transpose(1, 0, 2)
             .reshape(_NCH * _B * _CH))
    bits = lax.bitcast_convert_type(table, jnp.uint32) + jnp.uint32(0x8000)
    b3 = bits.reshape(-1, _D // 32, 2, _LANES)
    packed = (b3[:, :, 0, :] >> 16) | (b3[:, :, 1, :] & jnp.uint32(0xFFFF0000))
    tb = lax.bitcast_convert_type(packed.reshape(-1, _D // 2), jnp.int32)
    pos2 = pos_embed.reshape(_L, _D)
    mesh = plsc.VectorSubcoreMesh(core_axis_name="c", subcore_axis_name="s")
    run = pl.kernel(
        _body,
        out_type=jax.ShapeDtypeStruct((_B * _L, _D), jnp.float32),
        mesh=mesh,
        scratch_types=(
            [pltpu.VMEM((_BPW * _CH,), jnp.int32),
             pltpu.VMEM((_CH, _D), jnp.float32)]
            + [pltpu.VMEM((_CH, _D // 2), jnp.int32) for _ in range(_NG)]
            + [pltpu.VMEM((_CH, _D), jnp.float32) for _ in range(_NO)]
            + [pltpu.SemaphoreType.DMA] * (_NG + _NO)
        ),
    )
    out = run(idx_r, pos2, tb)
    return out.reshape(_B, _L, _D)


# speedup vs baseline: 1.8457x; 1.8457x over previous
import functools

import jax
import jax.numpy as jnp
from jax import lax
from jax.experimental import pallas as pl
from jax.experimental.pallas import tpu as pltpu
from jax.experimental.pallas import tpu_sc as plsc

_B = 4096
_L = 200
_D = 512
_CH = 40
_NCH = _L // _CH
_NC = 2
_NS = 16
_NW = _NC * _NS
_BPW = _B // _NW
_LANES = 16
_NG = 3
_NO = 2


def _body(idx_hbm, pos_hbm, table_hbm, out_hbm, idx_v, pos_v,
          gbuf0, gbuf1, gbuf2, obuf0, obuf1,
          gsem0, gsem1, gsem2, wsem0, wsem1):
    c = lax.axis_index("c")
    s = lax.axis_index("s")
    wid = s * _NC + c
    base = wid * _BPW
    gbufs = (gbuf0, gbuf1, gbuf2)
    obufs = (obuf0, obuf1)
    gsems = (gsem0, gsem1, gsem2)
    wsems = (wsem0, wsem1)

    def start_gather(bl, p):
        pltpu.async_copy(
            table_hbm.at[idx_v.at[pl.ds(bl * _CH, _CH)]], gbufs[p], gsems[p])

    def wait_gather(bl, p):
        pltpu.make_async_copy(
            table_hbm.at[idx_v.at[pl.ds(bl * _CH, _CH)]], gbufs[p],
            gsems[p]).wait()

    def out_slice(bl, ch):
        row0 = (base + bl) * _L + ch * _CH
        return out_hbm.at[pl.ds(row0, _CH)]

    def convert_add(gp, op):
        gbuf, obuf = gbufs[gp], obufs[op]
        shift = jnp.full((_LANES,), 16, dtype=jnp.int32)
        himask = jnp.full((_LANES,), -65536, dtype=jnp.int32)

        @plsc.parallel_loop(0, _CH, unroll=2)
        def r_body(r):
            for jj in range(_D // (2 * _LANES)):
                u = gbuf[r, pl.ds(jj * _LANES, _LANES)]
                lo = lax.bitcast_convert_type(u << shift, jnp.float32)
                hi = lax.bitcast_convert_type(u & himask, jnp.float32)
                sl0 = pl.ds(jj * 2 * _LANES, _LANES)
                sl1 = pl.ds(jj * 2 * _LANES + _LANES, _LANES)
                obuf[r, sl0] = lo + pos_v[r, sl0]
                obuf[r, sl1] = hi + pos_v[r, sl1]

    def start_write(bl, op, ch):
        pltpu.async_copy(obufs[op], out_slice(bl, ch), wsems[op])

    def wait_write(bl, op, ch):
        pltpu.make_async_copy(obufs[op], out_slice(bl, ch), wsems[op]).wait()

    def ch_body(ch, _):
        pltpu.sync_copy(
            idx_hbm.at[pl.ds(ch * _B * _CH + base * _CH, _BPW * _CH)], idx_v)
        pltpu.sync_copy(pos_hbm.at[pl.ds(ch * _CH, _CH)], pos_v)

        start_gather(0, 0)
        start_gather(1, 1)

        for bl in (0, 1):
            wait_gather(bl, bl)
            start_gather(bl + 2, bl + 2 - _NG * ((bl + 2) // _NG))
            convert_add(bl, bl)
            start_write(bl, bl, ch)

        def six_body(i, _):
            for k in range(6):
                bl = 6 * i + 2 + k
                gp = (2 + k) % _NG
                gq = (4 + k) % _NG
                op = k % _NO
                wait_gather(bl, gp)

                @pl.when(bl < _BPW - 2)
                def _():
                    start_gather(bl + 2, gq)

                wait_write(bl - 2, op, ch)
                convert_add(gp, op)
                start_write(bl, op, ch)
            return 0

        lax.fori_loop(0, (_BPW - 2) // 6, six_body, 0)

        wait_write(_BPW - 2, 0, ch)
        wait_write(_BPW - 1, 1, ch)
        return 0

    lax.fori_loop(0, _NCH, ch_body, 0)


@jax.jit
def kernel(inputs, table, pos_embed):
    idx_r = (inputs.astype(jnp.int32)
             .reshape(_B, _NCH, _CH)
             .transpose(1, 0, 2)
             .reshape(_NCH * _B * _CH))
    bits = lax.bitcast_convert_type(table, jnp.uint32) + jnp.uint32(0x8000)
    b3 = bits.reshape(-1, _D // 32, 2, _LANES)
    packed = (b3[:, :, 0, :] >> 16) | (b3[:, :, 1, :] & jnp.uint32(0xFFFF0000))
    tb = lax.bitcast_convert_type(packed.reshape(-1, _D // 2), jnp.int32)
    pos2 = pos_embed.reshape(_L, _D)
    mesh = plsc.VectorSubcoreMesh(core_axis_name="c", subcore_axis_name="s")
    run = pl.kernel(
        _body,
        out_type=jax.ShapeDtypeStruct((_B * _L, _D), jnp.float32),
        mesh=mesh,
        scratch_types=(
            [pltpu.VMEM((_BPW * _CH,), jnp.int32),
             pltpu.VMEM((_CH, _D), jnp.float32)]
            + [pltpu.VMEM((_CH, _D // 2), jnp.int32) for _ in range(_NG)]
            + [pltpu.VMEM((_CH, _D), jnp.float32) for _ in range(_NO)]
            + [pltpu.SemaphoreType.DMA] * (_NG + _NO)
        ),
    )
    out = run(idx_r, pos2, tb)
    return out.reshape(_B, _L, _D)
